# Initial kernel scaffold; baseline (speedup 1.0000x reference)
import jax
import jax.numpy as jnp
from jax.experimental import pallas as pl


def kernel(track_x, edge_pl_tr, edge_tr_ar, playlist_emb, artist_emb, type_emb, track_W, track_b, Wl0, bl0, Wr0, Wl1, bl1, Wr1):
    raise NotImplementedError("write your pallas kernel here")


# speedup vs baseline: 3.4277x; 3.4277x over previous
import functools

import jax
import jax.numpy as jnp
from jax import lax
from jax.experimental import pallas as pl
from jax.experimental.pallas import tpu as pltpu
from jax.experimental.pallas import tpu_sc as plsc

NUM_PL = 4000
NUM_TR = 4000
NUM_AR = 2000
HID = 128
N = NUM_PL + NUM_TR + NUM_AR
NPAD = 10240
DUMMY = N
NC, NS = 2, 16
TILES = NC * NS
BATCH = 128
ROWS_PER_TILE = NPAD // NS
CHUNK = 8


def _sc_body(compute_deg, x_hbm, src_hbm, dst_hbm, *refs):
    nb = src_hbm.shape[1]
    if compute_deg:
        (agg_out, deg_out, src_v, dst_v, rows_v, zbuf, agg_s, sem,
         deg_local) = refs
    else:
        agg_out, src_v, dst_v, rows_v, zbuf, agg_s, sem = refs
    cid = lax.axis_index("c")
    sid = lax.axis_index("s")
    wid = cid * NS + sid

    z16 = jnp.zeros((16,), jnp.float32)
    for i in range(8):
        for j in range(HID // 16):
            zbuf[i, pl.ds(j * 16, 16)] = z16

    @pl.loop(0, ROWS_PER_TILE // 8)
    def zero_agg(k):
        pltpu.sync_copy(zbuf, agg_s.at[pl.ds(sid * ROWS_PER_TILE + k * 8, 8)])

    if compute_deg:
        @pl.loop(0, NPAD // 16)
        def zero_deg(k):
            deg_local[pl.ds(k * 16, 16)] = z16

    plsc.subcore_barrier()

    @pl.loop(0, nb // CHUNK)
    def chunk_step(ch):
        pltpu.sync_copy(src_hbm.at[wid, pl.ds(ch * CHUNK, CHUNK)], src_v)
        pltpu.sync_copy(dst_hbm.at[wid, pl.ds(ch * CHUNK, CHUNK)], dst_v)

        @pl.loop(0, CHUNK)
        def edge_step(g):
            pltpu.async_copy(x_hbm.at[src_v.at[g]], rows_v, sem).wait()
            pltpu.sync_copy(rows_v, agg_s.at[dst_v.at[g]], add=True)
            if compute_deg:
                for j in range(BATCH // 16):
                    idx16 = dst_v[g, pl.ds(j * 16, 16)]
                    cnt, last = plsc.scan_count(idx16)
                    plsc.addupdate_scatter(
                        deg_local, [idx16], cnt.astype(jnp.float32),
                        mask=last)

    plsc.subcore_barrier()

    for blk in range(ROWS_PER_TILE // BATCH):
        sl = pl.ds(sid * ROWS_PER_TILE + blk * BATCH, BATCH)
        pltpu.sync_copy(agg_s.at[sl], rows_v)
        pltpu.sync_copy(rows_v, agg_out.at[cid, sl])
    if compute_deg:
        pltpu.sync_copy(deg_local, deg_out.at[cid, sid])


def _make_sc_segsum(nb, compute_deg):
    mesh = plsc.VectorSubcoreMesh(core_axis_name="c", subcore_axis_name="s",
                                  num_cores=NC, num_subcores=NS)
    out_type = [jax.ShapeDtypeStruct((NC, NPAD, HID), jnp.float32)]
    scratch = [
        pltpu.VMEM((CHUNK, BATCH), jnp.int32),
        pltpu.VMEM((CHUNK, BATCH), jnp.int32),
        pltpu.VMEM((BATCH, HID), jnp.float32),
        pltpu.VMEM((8, HID), jnp.float32),
        pltpu.VMEM_SHARED((NPAD, HID), jnp.float32),
        pltpu.SemaphoreType.DMA,
    ]
    if compute_deg:
        out_type.append(jax.ShapeDtypeStruct((NC, NS, NPAD), jnp.float32))
        scratch.append(pltpu.VMEM((NPAD,), jnp.float32))
    return pl.kernel(
        functools.partial(_sc_body, compute_deg),
        out_type=tuple(out_type),
        mesh=mesh,
        compiler_params=pltpu.CompilerParams(needs_layout_passes=False),
        scratch_types=scratch,
    )


def _x0_body(track_x, pl_emb, ar_emb, te, W, b, out):
    xtr = lax.dot_general(track_x[...], W[...], (((1,), (1,)), ((), ())),
                          preferred_element_type=jnp.float32)
    out[0:NUM_PL, :] = pl_emb[...] + te[0, :][None, :]
    out[NUM_PL:NUM_PL + NUM_TR, :] = xtr + b[0, :][None, :] + te[1, :][None, :]
    out[NUM_PL + NUM_TR:N, :] = ar_emb[...] + te[2, :][None, :]
    out[N:NPAD, :] = jnp.zeros((NPAD - N, HID), jnp.float32)


_x0_call = pl.pallas_call(
    _x0_body,
    out_shape=jax.ShapeDtypeStruct((NPAD, HID), jnp.float32),
)


def _layer_body(aggp, degp, x, Wl, bl, Wr, out):
    deg = jnp.sum(degp[...].reshape(NC * NS, NPAD), axis=0)[:, None]
    inv = 1.0 / jnp.maximum(deg, 1.0)
    agg = (aggp[0] + aggp[1]) * inv
    h = lax.dot_general(agg, Wl[...], (((1,), (1,)), ((), ())),
                        preferred_element_type=jnp.float32)
    h = h + bl[0, :][None, :]
    h = h + lax.dot_general(x[...], Wr[...], (((1,), (1,)), ((), ())),
                            preferred_element_type=jnp.float32)
    out[...] = jnp.maximum(h, 0.0)


_layer_call = pl.pallas_call(
    _layer_body,
    out_shape=jax.ShapeDtypeStruct((NPAD, HID), jnp.float32),
)


def kernel(track_x, edge_pl_tr, edge_tr_ar, playlist_emb, artist_emb,
           type_emb, track_W, track_b, Wl0, bl0, Wr0, Wl1, bl1, Wr1):
    e1s = edge_pl_tr[0].astype(jnp.int32)
    e1d = edge_pl_tr[1].astype(jnp.int32) + NUM_PL
    e2s = edge_tr_ar[0].astype(jnp.int32) + NUM_PL
    e2d = edge_tr_ar[1].astype(jnp.int32) + NUM_PL + NUM_TR
    n_edges = 2 * (e1s.shape[0] + e2s.shape[0])
    nb = -(-n_edges // (TILES * BATCH * CHUNK)) * CHUNK
    n_pad = TILES * nb * BATCH - n_edges
    fill = jnp.full((n_pad,), DUMMY, jnp.int32)
    src = jnp.concatenate([e1s, e1d, e2s, e2d, fill]).reshape(TILES, nb, BATCH)
    dst = jnp.concatenate([e1d, e1s, e2d, e2s, fill]).reshape(TILES, nb, BATCH)

    te = jnp.zeros((8, HID), jnp.float32).at[0:3].set(type_emb)
    tb = jnp.zeros((8, HID), jnp.float32).at[0].set(track_b)
    b0 = jnp.zeros((8, HID), jnp.float32).at[0].set(bl0)
    b1 = jnp.zeros((8, HID), jnp.float32).at[0].set(bl1)

    x0 = _x0_call(track_x, playlist_emb, artist_emb, te, track_W, tb)

    seg_deg = _make_sc_segsum(nb, True)
    seg = _make_sc_segsum(nb, False)

    agg0, degp = seg_deg(x0, src, dst)
    x1 = _layer_call(agg0, degp, x0, Wl0, b0, Wr0)
    agg1 = seg(x1, src, dst)
    if isinstance(agg1, (tuple, list)):
        agg1 = agg1[0]
    x2 = _layer_call(agg1, degp, x1, Wl1, b1, Wr1)

    return (x2[0:NUM_PL], x2[NUM_PL:NUM_PL + NUM_TR],
            x2[NUM_PL + NUM_TR:N])


# speedup vs baseline: 3.7989x; 1.1083x over previous
import functools

import jax
import jax.numpy as jnp
from jax import lax
from jax.experimental import pallas as pl
from jax.experimental.pallas import tpu as pltpu
from jax.experimental.pallas import tpu_sc as plsc

NUM_PL = 4000
NUM_TR = 4000
NUM_AR = 2000
HID = 128
N = NUM_PL + NUM_TR + NUM_AR
NPAD = 10240
DUMMY = N
NC, NS = 2, 16
TILES = NC * NS
BATCH = 128
ROWS_PER_TILE = NPAD // NS
CHUNK = 8


def _sc_body(compute_deg, x_hbm, src_hbm, dst_hbm, *refs):
    nb = src_hbm.shape[1]
    if compute_deg:
        (agg_out, deg_out, src_v, dst_v, rows_a, rows_b, zbuf, agg_s,
         sem_a, sem_b, deg_local) = refs
    else:
        agg_out, src_v, dst_v, rows_a, rows_b, zbuf, agg_s, sem_a, sem_b = refs
    rows_v = rows_a
    bufs = (rows_a, rows_b)
    sems = (sem_a, sem_b)
    cid = lax.axis_index("c")
    sid = lax.axis_index("s")
    wid = cid * NS + sid

    z16 = jnp.zeros((16,), jnp.float32)
    for i in range(8):
        for j in range(HID // 16):
            zbuf[i, pl.ds(j * 16, 16)] = z16

    @pl.loop(0, ROWS_PER_TILE // 8)
    def zero_agg(k):
        pltpu.sync_copy(zbuf, agg_s.at[pl.ds(sid * ROWS_PER_TILE + k * 8, 8)])

    if compute_deg:
        @pl.loop(0, NPAD // 16)
        def zero_deg(k):
            deg_local[pl.ds(k * 16, 16)] = z16

    plsc.subcore_barrier()

    @pl.loop(0, nb // CHUNK)
    def chunk_step(ch):
        pltpu.sync_copy(src_hbm.at[wid, pl.ds(ch * CHUNK, CHUNK)], src_v)
        pltpu.sync_copy(dst_hbm.at[wid, pl.ds(ch * CHUNK, CHUNK)], dst_v)

        copies = [None] * CHUNK
        copies[0] = pltpu.async_copy(x_hbm.at[src_v.at[0]], bufs[0], sems[0])
        for g in range(CHUNK):
            if g + 1 < CHUNK:
                copies[g + 1] = pltpu.async_copy(
                    x_hbm.at[src_v.at[g + 1]], bufs[(g + 1) % 2],
                    sems[(g + 1) % 2])
            copies[g].wait()
            pltpu.sync_copy(bufs[g % 2], agg_s.at[dst_v.at[g]], add=True)
            if compute_deg:
                for j in range(BATCH // 16):
                    idx16 = dst_v[g, pl.ds(j * 16, 16)]
                    cnt, last = plsc.scan_count(idx16)
                    plsc.addupdate_scatter(
                        deg_local, [idx16], cnt.astype(jnp.float32),
                        mask=last)

    plsc.subcore_barrier()

    for blk in range(ROWS_PER_TILE // BATCH):
        sl = pl.ds(sid * ROWS_PER_TILE + blk * BATCH, BATCH)
        pltpu.sync_copy(agg_s.at[sl], rows_v)
        pltpu.sync_copy(rows_v, agg_out.at[cid, sl])
    if compute_deg:
        pltpu.sync_copy(deg_local, deg_out.at[cid, sid])


def _make_sc_segsum(nb, compute_deg):
    mesh = plsc.VectorSubcoreMesh(core_axis_name="c", subcore_axis_name="s",
                                  num_cores=NC, num_subcores=NS)
    out_type = [jax.ShapeDtypeStruct((NC, NPAD, HID), jnp.float32)]
    scratch = [
        pltpu.VMEM((CHUNK, BATCH), jnp.int32),
        pltpu.VMEM((CHUNK, BATCH), jnp.int32),
        pltpu.VMEM((BATCH, HID), jnp.float32),
        pltpu.VMEM((BATCH, HID), jnp.float32),
        pltpu.VMEM((8, HID), jnp.float32),
        pltpu.VMEM_SHARED((NPAD, HID), jnp.float32),
        pltpu.SemaphoreType.DMA,
        pltpu.SemaphoreType.DMA,
    ]
    if compute_deg:
        out_type.append(jax.ShapeDtypeStruct((NC, NS, NPAD), jnp.float32))
        scratch.append(pltpu.VMEM((NPAD,), jnp.float32))
    return pl.kernel(
        functools.partial(_sc_body, compute_deg),
        out_type=tuple(out_type),
        mesh=mesh,
        compiler_params=pltpu.CompilerParams(needs_layout_passes=False),
        scratch_types=scratch,
    )


def _x0_body(track_x, pl_emb, ar_emb, te, W, b, out):
    xtr = lax.dot_general(track_x[...], W[...], (((1,), (1,)), ((), ())),
                          preferred_element_type=jnp.float32)
    out[0:NUM_PL, :] = pl_emb[...] + te[0, :][None, :]
    out[NUM_PL:NUM_PL + NUM_TR, :] = xtr + b[0, :][None, :] + te[1, :][None, :]
    out[NUM_PL + NUM_TR:N, :] = ar_emb[...] + te[2, :][None, :]
    out[N:NPAD, :] = jnp.zeros((NPAD - N, HID), jnp.float32)


_x0_call = pl.pallas_call(
    _x0_body,
    out_shape=jax.ShapeDtypeStruct((NPAD, HID), jnp.float32),
)


def _layer_body(aggp, degp, x, Wl, bl, Wr, out):
    deg = jnp.sum(degp[...].reshape(NC * NS, NPAD), axis=0)[:, None]
    inv = 1.0 / jnp.maximum(deg, 1.0)
    agg = (aggp[0] + aggp[1]) * inv
    h = lax.dot_general(agg, Wl[...], (((1,), (1,)), ((), ())),
                        preferred_element_type=jnp.float32)
    h = h + bl[0, :][None, :]
    h = h + lax.dot_general(x[...], Wr[...], (((1,), (1,)), ((), ())),
                            preferred_element_type=jnp.float32)
    out[...] = jnp.maximum(h, 0.0)


_layer_call = pl.pallas_call(
    _layer_body,
    out_shape=jax.ShapeDtypeStruct((NPAD, HID), jnp.float32),
)


def kernel(track_x, edge_pl_tr, edge_tr_ar, playlist_emb, artist_emb,
           type_emb, track_W, track_b, Wl0, bl0, Wr0, Wl1, bl1, Wr1):
    e1s = edge_pl_tr[0].astype(jnp.int32)
    e1d = edge_pl_tr[1].astype(jnp.int32) + NUM_PL
    e2s = edge_tr_ar[0].astype(jnp.int32) + NUM_PL
    e2d = edge_tr_ar[1].astype(jnp.int32) + NUM_PL + NUM_TR
    n_edges = 2 * (e1s.shape[0] + e2s.shape[0])
    nb = -(-n_edges // (TILES * BATCH * CHUNK)) * CHUNK
    n_pad = TILES * nb * BATCH - n_edges
    fill = jnp.full((n_pad,), DUMMY, jnp.int32)
    src = jnp.concatenate([e1s, e1d, e2s, e2d, fill]).reshape(TILES, nb, BATCH)
    dst = jnp.concatenate([e1d, e1s, e2d, e2s, fill]).reshape(TILES, nb, BATCH)

    te = jnp.zeros((8, HID), jnp.float32).at[0:3].set(type_emb)
    tb = jnp.zeros((8, HID), jnp.float32).at[0].set(track_b)
    b0 = jnp.zeros((8, HID), jnp.float32).at[0].set(bl0)
    b1 = jnp.zeros((8, HID), jnp.float32).at[0].set(bl1)

    x0 = _x0_call(track_x, playlist_emb, artist_emb, te, track_W, tb)

    seg_deg = _make_sc_segsum(nb, True)
    seg = _make_sc_segsum(nb, False)

    agg0, degp = seg_deg(x0, src, dst)
    x1 = _layer_call(agg0, degp, x0, Wl0, b0, Wr0)
    agg1 = seg(x1, src, dst)
    if isinstance(agg1, (tuple, list)):
        agg1 = agg1[0]
    x2 = _layer_call(agg1, degp, x1, Wl1, b1, Wr1)

    return (x2[0:NUM_PL], x2[NUM_PL:NUM_PL + NUM_TR],
            x2[NUM_PL + NUM_TR:N])


# speedup vs baseline: 4.3086x; 1.1342x over previous
import functools

import jax
import jax.numpy as jnp
from jax import lax
from jax.experimental import pallas as pl
from jax.experimental.pallas import tpu as pltpu
from jax.experimental.pallas import tpu_sc as plsc

NUM_PL = 4000
NUM_TR = 4000
NUM_AR = 2000
HID = 128
N = NUM_PL + NUM_TR + NUM_AR
NPAD = 10240
NC, NS = 2, 16
TILES = NC * NS
BATCH = 128
CHUNK = 8
RROWS = 6144
RDUMMY = 6000
ROWS_PER_TILE = RROWS // NS
SRC_DUMMY = N


def _sc_body(compute_deg, x_hbm, src_hbm, dst_hbm, *refs):
    nb = src_hbm.shape[1]
    if compute_deg:
        (agg_out, deg_out, src_v, dst_v, rows_a, rows_b, zbuf, agg_s,
         sem_a, sem_b, deg_local) = refs
    else:
        agg_out, src_v, dst_v, rows_a, rows_b, zbuf, agg_s, sem_a, sem_b = refs
    bufs = (rows_a, rows_b)
    sems = (sem_a, sem_b)
    cid = lax.axis_index("c")
    sid = lax.axis_index("s")
    wid = cid * NS + sid

    z16 = jnp.zeros((16,), jnp.float32)
    for i in range(8):
        for j in range(HID // 16):
            zbuf[i, pl.ds(j * 16, 16)] = z16

    @pl.loop(0, ROWS_PER_TILE // 8)
    def zero_agg(k):
        pltpu.sync_copy(zbuf, agg_s.at[pl.ds(sid * ROWS_PER_TILE + k * 8, 8)])

    if compute_deg:
        @pl.loop(0, RROWS // 16)
        def zero_deg(k):
            deg_local[pl.ds(k * 16, 16)] = z16

    plsc.subcore_barrier()

    @pl.loop(0, nb // CHUNK)
    def chunk_step(ch):
        pltpu.sync_copy(src_hbm.at[wid, pl.ds(ch * CHUNK, CHUNK)], src_v)
        pltpu.sync_copy(dst_hbm.at[wid, pl.ds(ch * CHUNK, CHUNK)], dst_v)

        copies = [None] * CHUNK
        copies[0] = pltpu.async_copy(x_hbm.at[src_v.at[0]], bufs[0], sems[0])
        for g in range(CHUNK):
            if g + 1 < CHUNK:
                copies[g + 1] = pltpu.async_copy(
                    x_hbm.at[src_v.at[g + 1]], bufs[(g + 1) % 2],
                    sems[(g + 1) % 2])
            copies[g].wait()
            pltpu.sync_copy(bufs[g % 2], agg_s.at[dst_v.at[g]], add=True)
            if compute_deg:
                for j in range(BATCH // 16):
                    idx16 = dst_v[g, pl.ds(j * 16, 16)]
                    cnt, last = plsc.scan_count(idx16)
                    plsc.addupdate_scatter(
                        deg_local, [idx16], cnt.astype(jnp.float32),
                        mask=last)

    plsc.subcore_barrier()

    for blk in range(ROWS_PER_TILE // BATCH):
        sl = pl.ds(sid * ROWS_PER_TILE + blk * BATCH, BATCH)
        pltpu.sync_copy(agg_s.at[sl], rows_a)
        pltpu.sync_copy(rows_a, agg_out.at[cid, sl])
    if compute_deg:
        pltpu.sync_copy(deg_local, deg_out.at[cid, sid])


def _make_sc_segsum(nb, compute_deg):
    mesh = plsc.VectorSubcoreMesh(core_axis_name="c", subcore_axis_name="s",
                                  num_cores=NC, num_subcores=NS)
    out_type = [jax.ShapeDtypeStruct((NC, RROWS, HID), jnp.float32)]
    scratch = [
        pltpu.VMEM((CHUNK, BATCH), jnp.int32),
        pltpu.VMEM((CHUNK, BATCH), jnp.int32),
        pltpu.VMEM((BATCH, HID), jnp.float32),
        pltpu.VMEM((BATCH, HID), jnp.float32),
        pltpu.VMEM((8, HID), jnp.float32),
        pltpu.VMEM_SHARED((RROWS, HID), jnp.float32),
        pltpu.SemaphoreType.DMA,
        pltpu.SemaphoreType.DMA,
    ]
    if compute_deg:
        out_type.append(jax.ShapeDtypeStruct((NC, NS, RROWS), jnp.float32))
        scratch.append(pltpu.VMEM((RROWS,), jnp.float32))
    return pl.kernel(
        functools.partial(_sc_body, compute_deg),
        out_type=tuple(out_type),
        mesh=mesh,
        compiler_params=pltpu.CompilerParams(needs_layout_passes=False),
        scratch_types=scratch,
    )


def _x0_body(track_x, pl_emb, ar_emb, te, W, b, out):
    xtr = lax.dot_general(track_x[...], W[...], (((1,), (1,)), ((), ())),
                          preferred_element_type=jnp.float32)
    out[0:NUM_PL, :] = pl_emb[...] + te[0, :][None, :]
    out[NUM_PL:NUM_PL + NUM_TR, :] = xtr + b[0, :][None, :] + te[1, :][None, :]
    out[NUM_PL + NUM_TR:N, :] = ar_emb[...] + te[2, :][None, :]
    out[N:NPAD, :] = jnp.zeros((NPAD - N, HID), jnp.float32)


_x0_call = pl.pallas_call(
    _x0_body,
    out_shape=jax.ShapeDtypeStruct((NPAD, HID), jnp.float32),
)


def _layer_body(aggp, degp, x, Wl, bl, Wr, out):
    agg = jnp.concatenate([
        aggp[0, 0:NUM_PL],
        aggp[1, 0:NUM_TR],
        aggp[0, NUM_PL:NUM_PL + NUM_AR],
        jnp.zeros((NPAD - N, HID), jnp.float32),
    ], axis=0)
    deg0 = jnp.sum(degp[0], axis=0)
    deg1 = jnp.sum(degp[1], axis=0)
    deg = jnp.concatenate([
        deg0[0:NUM_PL],
        deg1[0:NUM_TR],
        deg0[NUM_PL:NUM_PL + NUM_AR],
        jnp.ones((NPAD - N,), jnp.float32),
    ])[:, None]
    inv = 1.0 / jnp.maximum(deg, 1.0)
    h = lax.dot_general(agg * inv, Wl[...], (((1,), (1,)), ((), ())),
                        preferred_element_type=jnp.float32)
    h = h + bl[0, :][None, :]
    h = h + lax.dot_general(x[...], Wr[...], (((1,), (1,)), ((), ())),
                            preferred_element_type=jnp.float32)
    out[...] = jnp.maximum(h, 0.0)


_layer_call = pl.pallas_call(
    _layer_body,
    out_shape=jax.ShapeDtypeStruct((NPAD, HID), jnp.float32),
)


def kernel(track_x, edge_pl_tr, edge_tr_ar, playlist_emb, artist_emb,
           type_emb, track_W, track_b, Wl0, bl0, Wr0, Wl1, bl1, Wr1):
    e1p = edge_pl_tr[0].astype(jnp.int32)
    e1t = edge_pl_tr[1].astype(jnp.int32)
    e2t = edge_tr_ar[0].astype(jnp.int32)
    e2a = edge_tr_ar[1].astype(jnp.int32)
    n_half = e1p.shape[0] + e2a.shape[0]
    nb = -(-n_half // (NS * BATCH * CHUNK)) * CHUNK
    n_pad = NS * nb * BATCH - n_half
    sfill = jnp.full((n_pad,), SRC_DUMMY, jnp.int32)
    dfill = jnp.full((n_pad,), RDUMMY, jnp.int32)
    src0 = jnp.concatenate([e1t + NUM_PL, e2t + NUM_PL, sfill])
    dst0 = jnp.concatenate([e1p, e2a + NUM_PL, dfill])
    src1 = jnp.concatenate([e1p, e2a + NUM_PL + NUM_TR, sfill])
    dst1 = jnp.concatenate([e1t, e2t, dfill])
    src = jnp.concatenate([src0, src1]).reshape(TILES, nb, BATCH)
    dst = jnp.concatenate([dst0, dst1]).reshape(TILES, nb, BATCH)

    te = jnp.zeros((8, HID), jnp.float32).at[0:3].set(type_emb)
    tb = jnp.zeros((8, HID), jnp.float32).at[0].set(track_b)
    b0 = jnp.zeros((8, HID), jnp.float32).at[0].set(bl0)
    b1 = jnp.zeros((8, HID), jnp.float32).at[0].set(bl1)

    x0 = _x0_call(track_x, playlist_emb, artist_emb, te, track_W, tb)

    seg_deg = _make_sc_segsum(nb, True)
    seg = _make_sc_segsum(nb, False)

    agg0, degp = seg_deg(x0, src, dst)
    x1 = _layer_call(agg0, degp, x0, Wl0, b0, Wr0)
    agg1 = seg(x1, src, dst)
    if isinstance(agg1, (tuple, list)):
        agg1 = agg1[0]
    x2 = _layer_call(agg1, degp, x1, Wl1, b1, Wr1)

    return (x2[0:NUM_PL], x2[NUM_PL:NUM_PL + NUM_TR],
            x2[NUM_PL + NUM_TR:N])


# speedup vs baseline: 4.3280x; 1.0045x over previous
import functools

import jax
import jax.numpy as jnp
from jax import lax
from jax.experimental import pallas as pl
from jax.experimental.pallas import tpu as pltpu
from jax.experimental.pallas import tpu_sc as plsc

NUM_PL = 4000
NUM_TR = 4000
NUM_AR = 2000
HID = 128
N = NUM_PL + NUM_TR + NUM_AR
NPAD = 10240
NC, NS = 2, 16
TILES = NC * NS
BATCH = 128
CHUNK = 8
NBUF = 4
RROWS = 6144
RDUMMY = 6000
ROWS_PER_TILE = RROWS // NS
SRC_DUMMY = N


def _sc_body(compute_deg, x_hbm, src_hbm, dst_hbm, *refs):
    nb = src_hbm.shape[1]
    if compute_deg:
        (agg_out, deg_out, src_v, dst_v, b0, b1, b2, b3, zbuf, agg_s,
         g0, g1, g2, g3, s0, s1, s2, s3, deg_local) = refs
    else:
        (agg_out, src_v, dst_v, b0, b1, b2, b3, zbuf, agg_s,
         g0, g1, g2, g3, s0, s1, s2, s3) = refs
    bufs = (b0, b1, b2, b3)
    gsems = (g0, g1, g2, g3)
    ssems = (s0, s1, s2, s3)
    cid = lax.axis_index("c")
    sid = lax.axis_index("s")
    wid = cid * NS + sid

    z16 = jnp.zeros((16,), jnp.float32)
    for i in range(8):
        for j in range(HID // 16):
            zbuf[i, pl.ds(j * 16, 16)] = z16

    @pl.loop(0, ROWS_PER_TILE // 8)
    def zero_agg(k):
        pltpu.sync_copy(zbuf, agg_s.at[pl.ds(sid * ROWS_PER_TILE + k * 8, 8)])

    if compute_deg:
        @pl.loop(0, RROWS // 16)
        def zero_deg(k):
            deg_local[pl.ds(k * 16, 16)] = z16

    plsc.subcore_barrier()

    @pl.loop(0, nb // CHUNK)
    def chunk_step(ch):
        pltpu.sync_copy(src_hbm.at[wid, pl.ds(ch * CHUNK, CHUNK)], src_v)
        pltpu.sync_copy(dst_hbm.at[wid, pl.ds(ch * CHUNK, CHUNK)], dst_v)

        copies = [None] * CHUNK
        scats = [None] * CHUNK
        for g in range(NBUF):
            copies[g] = pltpu.async_copy(
                x_hbm.at[src_v.at[g]], bufs[g], gsems[g])
        for g in range(CHUNK):
            copies[g].wait()
            scats[g] = pltpu.async_copy(
                bufs[g % NBUF], agg_s.at[dst_v.at[g]], ssems[g % NBUF],
                add=True)
            ng = g + NBUF
            if ng < CHUNK:
                scats[g].wait()
                copies[ng] = pltpu.async_copy(
                    x_hbm.at[src_v.at[ng]], bufs[g % NBUF], gsems[g % NBUF])
            if compute_deg:
                for j in range(BATCH // 16):
                    idx16 = dst_v[g, pl.ds(j * 16, 16)]
                    cnt, last = plsc.scan_count(idx16)
                    plsc.addupdate_scatter(
                        deg_local, [idx16], cnt.astype(jnp.float32),
                        mask=last)
        for g in range(CHUNK - NBUF, CHUNK):
            scats[g].wait()

    plsc.subcore_barrier()

    for blk in range(ROWS_PER_TILE // BATCH):
        sl = pl.ds(sid * ROWS_PER_TILE + blk * BATCH, BATCH)
        pltpu.sync_copy(agg_s.at[sl], bufs[blk % NBUF])
        pltpu.sync_copy(bufs[blk % NBUF], agg_out.at[cid, sl])
    if compute_deg:
        pltpu.sync_copy(deg_local, deg_out.at[cid, sid])


def _make_sc_segsum(nb, compute_deg):
    mesh = plsc.VectorSubcoreMesh(core_axis_name="c", subcore_axis_name="s",
                                  num_cores=NC, num_subcores=NS)
    out_type = [jax.ShapeDtypeStruct((NC, RROWS, HID), jnp.float32)]
    scratch = [
        pltpu.VMEM((CHUNK, BATCH), jnp.int32),
        pltpu.VMEM((CHUNK, BATCH), jnp.int32),
        pltpu.VMEM((BATCH, HID), jnp.float32),
        pltpu.VMEM((BATCH, HID), jnp.float32),
        pltpu.VMEM((BATCH, HID), jnp.float32),
        pltpu.VMEM((BATCH, HID), jnp.float32),
        pltpu.VMEM((8, HID), jnp.float32),
        pltpu.VMEM_SHARED((RROWS, HID), jnp.float32),
    ] + [pltpu.SemaphoreType.DMA] * 8
    if compute_deg:
        out_type.append(jax.ShapeDtypeStruct((NC, NS, RROWS), jnp.float32))
        scratch.append(pltpu.VMEM((RROWS,), jnp.float32))
    return pl.kernel(
        functools.partial(_sc_body, compute_deg),
        out_type=tuple(out_type),
        mesh=mesh,
        compiler_params=pltpu.CompilerParams(needs_layout_passes=False),
        scratch_types=scratch,
    )


def _x0_body(track_x, pl_emb, ar_emb, te, W, b, out):
    xtr = lax.dot_general(track_x[...], W[...], (((1,), (1,)), ((), ())),
                          preferred_element_type=jnp.float32)
    out[0:NUM_PL, :] = pl_emb[...] + te[0, :][None, :]
    out[NUM_PL:NUM_PL + NUM_TR, :] = xtr + b[0, :][None, :] + te[1, :][None, :]
    out[NUM_PL + NUM_TR:N, :] = ar_emb[...] + te[2, :][None, :]
    out[N:NPAD, :] = jnp.zeros((NPAD - N, HID), jnp.float32)


_x0_call = pl.pallas_call(
    _x0_body,
    out_shape=jax.ShapeDtypeStruct((NPAD, HID), jnp.float32),
)


def _layer_body(aggp, degp, x, Wl, bl, Wr, out):
    agg = jnp.concatenate([
        aggp[0, 0:NUM_PL],
        aggp[1, 0:NUM_TR],
        aggp[0, NUM_PL:NUM_PL + NUM_AR],
        jnp.zeros((NPAD - N, HID), jnp.float32),
    ], axis=0)
    deg0 = jnp.sum(degp[0], axis=0)
    deg1 = jnp.sum(degp[1], axis=0)
    deg = jnp.concatenate([
        deg0[0:NUM_PL],
        deg1[0:NUM_TR],
        deg0[NUM_PL:NUM_PL + NUM_AR],
        jnp.ones((NPAD - N,), jnp.float32),
    ])[:, None]
    inv = 1.0 / jnp.maximum(deg, 1.0)
    h = lax.dot_general(agg * inv, Wl[...], (((1,), (1,)), ((), ())),
                        preferred_element_type=jnp.float32)
    h = h + bl[0, :][None, :]
    h = h + lax.dot_general(x[...], Wr[...], (((1,), (1,)), ((), ())),
                            preferred_element_type=jnp.float32)
    out[...] = jnp.maximum(h, 0.0)


_layer_call = pl.pallas_call(
    _layer_body,
    out_shape=jax.ShapeDtypeStruct((NPAD, HID), jnp.float32),
)


def kernel(track_x, edge_pl_tr, edge_tr_ar, playlist_emb, artist_emb,
           type_emb, track_W, track_b, Wl0, bl0, Wr0, Wl1, bl1, Wr1):
    e1p = edge_pl_tr[0].astype(jnp.int32)
    e1t = edge_pl_tr[1].astype(jnp.int32)
    e2t = edge_tr_ar[0].astype(jnp.int32)
    e2a = edge_tr_ar[1].astype(jnp.int32)
    n_half = e1p.shape[0] + e2a.shape[0]
    nb = -(-n_half // (NS * BATCH * CHUNK)) * CHUNK
    n_pad = NS * nb * BATCH - n_half
    sfill = jnp.full((n_pad,), SRC_DUMMY, jnp.int32)
    dfill = jnp.full((n_pad,), RDUMMY, jnp.int32)
    src0 = jnp.concatenate([e1t + NUM_PL, e2t + NUM_PL, sfill])
    dst0 = jnp.concatenate([e1p, e2a + NUM_PL, dfill])
    src1 = jnp.concatenate([e1p, e2a + NUM_PL + NUM_TR, sfill])
    dst1 = jnp.concatenate([e1t, e2t, dfill])
    src = jnp.concatenate([src0, src1]).reshape(TILES, nb, BATCH)
    dst = jnp.concatenate([dst0, dst1]).reshape(TILES, nb, BATCH)

    te = jnp.zeros((8, HID), jnp.float32).at[0:3].set(type_emb)
    tb = jnp.zeros((8, HID), jnp.float32).at[0].set(track_b)
    b0 = jnp.zeros((8, HID), jnp.float32).at[0].set(bl0)
    b1 = jnp.zeros((8, HID), jnp.float32).at[0].set(bl1)

    x0 = _x0_call(track_x, playlist_emb, artist_emb, te, track_W, tb)

    seg_deg = _make_sc_segsum(nb, True)
    seg = _make_sc_segsum(nb, False)

    agg0, degp = seg_deg(x0, src, dst)
    x1 = _layer_call(agg0, degp, x0, Wl0, b0, Wr0)
    agg1 = seg(x1, src, dst)
    if isinstance(agg1, (tuple, list)):
        agg1 = agg1[0]
    x2 = _layer_call(agg1, degp, x1, Wl1, b1, Wr1)

    return (x2[0:NUM_PL], x2[NUM_PL:NUM_PL + NUM_TR],
            x2[NUM_PL + NUM_TR:N])


# speedup vs baseline: 4.4370x; 1.0252x over previous
import functools

import jax
import jax.numpy as jnp
from jax import lax
from jax.experimental import pallas as pl
from jax.experimental.pallas import tpu as pltpu
from jax.experimental.pallas import tpu_sc as plsc

NUM_PL = 4000
NUM_TR = 4000
NUM_AR = 2000
HID = 128
N = NUM_PL + NUM_TR + NUM_AR
NPAD = 10240
NC, NS = 2, 16
TILES = NC * NS
BATCH = 128
CHUNK = 16
NBUF = 4
RROWS = 6144
RDUMMY = 6000
ROWS_PER_TILE = RROWS // NS
SRC_DUMMY = N


def _sc_body(compute_deg, x_hbm, src_hbm, dst_hbm, *refs):
    nb = src_hbm.shape[1]
    if compute_deg:
        (agg_out, deg_out, src_v, dst_v, b0, b1, b2, b3, zbuf, agg_s,
         g0, g1, g2, g3, s0, s1, s2, s3, deg_local) = refs
    else:
        (agg_out, src_v, dst_v, b0, b1, b2, b3, zbuf, agg_s,
         g0, g1, g2, g3, s0, s1, s2, s3) = refs
    bufs = (b0, b1, b2, b3)
    gsems = (g0, g1, g2, g3)
    ssems = (s0, s1, s2, s3)
    cid = lax.axis_index("c")
    sid = lax.axis_index("s")
    wid = cid * NS + sid

    z16 = jnp.zeros((16,), jnp.float32)
    for i in range(8):
        for j in range(HID // 16):
            zbuf[i, pl.ds(j * 16, 16)] = z16

    @pl.loop(0, ROWS_PER_TILE // 8)
    def zero_agg(k):
        pltpu.sync_copy(zbuf, agg_s.at[pl.ds(sid * ROWS_PER_TILE + k * 8, 8)])

    if compute_deg:
        @pl.loop(0, RROWS // 16)
        def zero_deg(k):
            deg_local[pl.ds(k * 16, 16)] = z16

    plsc.subcore_barrier()

    @pl.loop(0, nb // CHUNK)
    def chunk_step(ch):
        pltpu.sync_copy(src_hbm.at[wid, pl.ds(ch * CHUNK, CHUNK)], src_v)
        pltpu.sync_copy(dst_hbm.at[wid, pl.ds(ch * CHUNK, CHUNK)], dst_v)

        copies = [None] * CHUNK
        scats = [None] * CHUNK
        for g in range(NBUF):
            copies[g] = pltpu.async_copy(
                x_hbm.at[src_v.at[g]], bufs[g], gsems[g])
        for g in range(CHUNK):
            copies[g].wait()
            scats[g] = pltpu.async_copy(
                bufs[g % NBUF], agg_s.at[dst_v.at[g]], ssems[g % NBUF],
                add=True)
            ng = g + NBUF
            if ng < CHUNK:
                scats[g].wait()
                copies[ng] = pltpu.async_copy(
                    x_hbm.at[src_v.at[ng]], bufs[g % NBUF], gsems[g % NBUF])
            if compute_deg:
                for j in range(BATCH // 16):
                    idx16 = dst_v[g, pl.ds(j * 16, 16)]
                    cnt, last = plsc.scan_count(idx16)
                    plsc.addupdate_scatter(
                        deg_local, [idx16], cnt.astype(jnp.float32),
                        mask=last)
        for g in range(CHUNK - NBUF, CHUNK):
            scats[g].wait()

    plsc.subcore_barrier()

    for blk in range(ROWS_PER_TILE // BATCH):
        sl = pl.ds(sid * ROWS_PER_TILE + blk * BATCH, BATCH)
        pltpu.sync_copy(agg_s.at[sl], bufs[blk % NBUF])
        pltpu.sync_copy(bufs[blk % NBUF], agg_out.at[cid, sl])
    if compute_deg:
        pltpu.sync_copy(deg_local, deg_out.at[cid, sid])


def _make_sc_segsum(nb, compute_deg):
    mesh = plsc.VectorSubcoreMesh(core_axis_name="c", subcore_axis_name="s",
                                  num_cores=NC, num_subcores=NS)
    out_type = [jax.ShapeDtypeStruct((NC, RROWS, HID), jnp.float32)]
    scratch = [
        pltpu.VMEM((CHUNK, BATCH), jnp.int32),
        pltpu.VMEM((CHUNK, BATCH), jnp.int32),
        pltpu.VMEM((BATCH, HID), jnp.float32),
        pltpu.VMEM((BATCH, HID), jnp.float32),
        pltpu.VMEM((BATCH, HID), jnp.float32),
        pltpu.VMEM((BATCH, HID), jnp.float32),
        pltpu.VMEM((8, HID), jnp.float32),
        pltpu.VMEM_SHARED((RROWS, HID), jnp.float32),
    ] + [pltpu.SemaphoreType.DMA] * 8
    if compute_deg:
        out_type.append(jax.ShapeDtypeStruct((NC, NS, RROWS), jnp.float32))
        scratch.append(pltpu.VMEM((RROWS,), jnp.float32))
    return pl.kernel(
        functools.partial(_sc_body, compute_deg),
        out_type=tuple(out_type),
        mesh=mesh,
        compiler_params=pltpu.CompilerParams(needs_layout_passes=False),
        scratch_types=scratch,
    )


def _x0_body(track_x, pl_emb, ar_emb, te, W, b, out):
    xtr = lax.dot_general(track_x[...], W[...], (((1,), (1,)), ((), ())),
                          preferred_element_type=jnp.float32)
    out[0:NUM_PL, :] = pl_emb[...] + te[0, :][None, :]
    out[NUM_PL:NUM_PL + NUM_TR, :] = xtr + b[0, :][None, :] + te[1, :][None, :]
    out[NUM_PL + NUM_TR:N, :] = ar_emb[...] + te[2, :][None, :]
    out[N:NPAD, :] = jnp.zeros((NPAD - N, HID), jnp.float32)


_x0_call = pl.pallas_call(
    _x0_body,
    out_shape=jax.ShapeDtypeStruct((NPAD, HID), jnp.float32),
)


def _layer_body(aggp, degp, x, Wl, bl, Wr, out):
    agg = jnp.concatenate([
        aggp[0, 0:NUM_PL],
        aggp[1, 0:NUM_TR],
        aggp[0, NUM_PL:NUM_PL + NUM_AR],
        jnp.zeros((NPAD - N, HID), jnp.float32),
    ], axis=0)
    deg0 = jnp.sum(degp[0], axis=0)
    deg1 = jnp.sum(degp[1], axis=0)
    deg = jnp.concatenate([
        deg0[0:NUM_PL],
        deg1[0:NUM_TR],
        deg0[NUM_PL:NUM_PL + NUM_AR],
        jnp.ones((NPAD - N,), jnp.float32),
    ])[:, None]
    inv = 1.0 / jnp.maximum(deg, 1.0)
    h = lax.dot_general(agg * inv, Wl[...], (((1,), (1,)), ((), ())),
                        preferred_element_type=jnp.float32)
    h = h + bl[0, :][None, :]
    h = h + lax.dot_general(x[...], Wr[...], (((1,), (1,)), ((), ())),
                            preferred_element_type=jnp.float32)
    out[...] = jnp.maximum(h, 0.0)


_layer_call = pl.pallas_call(
    _layer_body,
    out_shape=jax.ShapeDtypeStruct((NPAD, HID), jnp.float32),
)


def kernel(track_x, edge_pl_tr, edge_tr_ar, playlist_emb, artist_emb,
           type_emb, track_W, track_b, Wl0, bl0, Wr0, Wl1, bl1, Wr1):
    e1p = edge_pl_tr[0].astype(jnp.int32)
    e1t = edge_pl_tr[1].astype(jnp.int32)
    e2t = edge_tr_ar[0].astype(jnp.int32)
    e2a = edge_tr_ar[1].astype(jnp.int32)
    n_half = e1p.shape[0] + e2a.shape[0]
    nb = -(-n_half // (NS * BATCH * CHUNK)) * CHUNK
    n_pad = NS * nb * BATCH - n_half
    sfill = jnp.full((n_pad,), SRC_DUMMY, jnp.int32)
    dfill = jnp.full((n_pad,), RDUMMY, jnp.int32)
    src0 = jnp.concatenate([e1t + NUM_PL, e2t + NUM_PL, sfill])
    dst0 = jnp.concatenate([e1p, e2a + NUM_PL, dfill])
    src1 = jnp.concatenate([e1p, e2a + NUM_PL + NUM_TR, sfill])
    dst1 = jnp.concatenate([e1t, e2t, dfill])
    src = jnp.concatenate([src0, src1]).reshape(TILES, nb, BATCH)
    dst = jnp.concatenate([dst0, dst1]).reshape(TILES, nb, BATCH)

    te = jnp.zeros((8, HID), jnp.float32).at[0:3].set(type_emb)
    tb = jnp.zeros((8, HID), jnp.float32).at[0].set(track_b)
    b0 = jnp.zeros((8, HID), jnp.float32).at[0].set(bl0)
    b1 = jnp.zeros((8, HID), jnp.float32).at[0].set(bl1)

    x0 = _x0_call(track_x, playlist_emb, artist_emb, te, track_W, tb)

    seg_deg = _make_sc_segsum(nb, True)
    seg = _make_sc_segsum(nb, False)

    agg0, degp = seg_deg(x0, src, dst)
    x1 = _layer_call(agg0, degp, x0, Wl0, b0, Wr0)
    agg1 = seg(x1, src, dst)
    if isinstance(agg1, (tuple, list)):
        agg1 = agg1[0]
    x2 = _layer_call(agg1, degp, x1, Wl1, b1, Wr1)

    return (x2[0:NUM_PL], x2[NUM_PL:NUM_PL + NUM_TR],
            x2[NUM_PL + NUM_TR:N])
